# Initial kernel scaffold; baseline (speedup 1.0000x reference)
import jax
import jax.numpy as jnp
from jax.experimental import pallas as pl


def kernel(X, edges, W, W0, inv_norm_constant):
    raise NotImplementedError("write your pallas kernel here")


# speedup vs baseline: 1.9755x; 1.9755x over previous
import functools

import jax
import jax.numpy as jnp
from jax import lax
from jax.experimental import pallas as pl
from jax.experimental.pallas import tpu as pltpu
from jax.experimental.pallas import tpu_sc as plsc

_N = 10000
_D = 128
_R = 8
_NT = 16
_ROWS = 8
_RW = 80
_CH = _ROWS * _RW
_NP = _NT * _CH
_TBL = 80008
_RB = 2000


def _sc_node_acc(dst_hbm, rel_hbm, tbl_hbm, out_hbm,
                 dst_v, rel_v, idx_v, vals_v, zbuf, obuf, acc, sem):
    s = lax.axis_index("s")
    for i in range(_CH // 16):
        zbuf[pl.ds(i * 16, 16)] = jnp.zeros((16,), jnp.float32)
    pltpu.sync_copy(zbuf, acc.at[pl.ds(s * _CH, _CH)])
    pltpu.sync_copy(dst_hbm.at[s], dst_v)
    pltpu.sync_copy(rel_hbm.at[s], rel_v)
    for j in range(_ROWS):
        for i in range(_RW // 16):
            sl = pl.ds(i * 16, 16)
            idx_v[j, sl] = dst_v[j, sl] * _R + rel_v[j, sl]
    for j in range(_ROWS):
        pltpu.async_copy(tbl_hbm.at[idx_v.at[j]], vals_v.at[j], sem).wait()
    plsc.subcore_barrier()
    for j in range(_ROWS):
        pltpu.sync_copy(vals_v.at[j], acc.at[dst_v.at[j]], add=True)
    plsc.subcore_barrier()
    pltpu.sync_copy(acc.at[pl.ds(s * _CH, _CH)], obuf)
    pltpu.sync_copy(obuf, out_hbm.at[pl.ds(s * _CH, _CH)])


_sc_kernel = functools.partial(
    pl.kernel,
    out_type=jax.ShapeDtypeStruct((_NP,), jnp.float32),
    mesh=plsc.VectorSubcoreMesh(
        core_axis_name="c", subcore_axis_name="s", num_cores=1),
    scratch_types=[
        pltpu.VMEM((_ROWS, _RW), jnp.int32),
        pltpu.VMEM((_ROWS, _RW), jnp.int32),
        pltpu.VMEM((_ROWS, _RW), jnp.int32),
        pltpu.VMEM((_ROWS, _RW), jnp.float32),
        pltpu.VMEM((_CH,), jnp.float32),
        pltpu.VMEM((_CH,), jnp.float32),
        pltpu.VMEM_SHARED((_NP,), jnp.float32),
        pltpu.SemaphoreType.DMA,
    ],
)(_sc_node_acc)


def _tc_body(x_ref, w_ref, a_ref, o_ref):
    o_ref[0] = (
        jnp.dot(x_ref[0], w_ref[...], preferred_element_type=jnp.float32)
        + a_ref[...]
    )


def kernel(X, edges, W, W0, inv_norm_constant):
    b, n, d = X.shape
    dst = edges[1, :n]
    rel = edges[2, :n]
    pad = _NP - n
    dst_p = jnp.concatenate(
        [dst, jnp.full((pad,), n, jnp.int32)]).reshape(_NT, _ROWS, _RW)
    rel_p = jnp.concatenate(
        [rel, jnp.zeros((pad,), jnp.int32)]).reshape(_NT, _ROWS, _RW)
    tbl = jnp.concatenate([
        inv_norm_constant.reshape(-1),
        jnp.zeros((_TBL - n * _R,), jnp.float32),
    ])
    node_acc = _sc_kernel(dst_p, rel_p, tbl)
    acc_col = node_acc[:n, None]

    out = pl.pallas_call(
        _tc_body,
        out_shape=jax.ShapeDtypeStruct((b, n, d), jnp.float32),
        grid=(n // _RB,),
        in_specs=[
            pl.BlockSpec((1, _RB, d), lambda i: (0, i, 0)),
            pl.BlockSpec((d, d), lambda i: (0, 0)),
            pl.BlockSpec((_RB, 1), lambda i: (i, 0)),
        ],
        out_specs=pl.BlockSpec((1, _RB, d), lambda i: (0, i, 0)),
    )(X, W0, acc_col)
    return out


# speedup vs baseline: 2.3461x; 1.1876x over previous
import functools

import jax
import jax.numpy as jnp
from jax import lax
from jax.experimental import pallas as pl
from jax.experimental.pallas import tpu as pltpu
from jax.experimental.pallas import tpu_sc as plsc

_N = 10000
_D = 128
_R = 8
_NT = 16
_ROWS = 8
_RW = 80
_CH = _ROWS * _RW
_NP = _NT * _CH
_TAILR = (_N - (_NT - 1) * _CH) // _RW
_RB = 2000


def _sc_node_acc(edges_hbm, tbl_hbm, out_hbm,
                 dst_s, rel_s, dst2, idx2, vals, zbuf, obuf, acc, sem):
    s = lax.axis_index("s")
    for i in range(_CH // 16):
        zbuf[pl.ds(i * 16, 16)] = jnp.zeros((16,), jnp.float32)
    pltpu.sync_copy(zbuf, acc.at[pl.ds(s * _CH, _CH)])
    cp_d = pltpu.async_copy(
        edges_hbm.at[pl.ds(1, 1), pl.ds(s * _CH, _CH)], dst_s, sem)
    cp_r = pltpu.async_copy(
        edges_hbm.at[pl.ds(2, 1), pl.ds(s * _CH, _CH)], rel_s, sem)
    cp_d.wait()
    cp_r.wait()
    gathers = []
    for j in range(_ROWS):
        for i in range(_RW // 16):
            sl = pl.ds(j * _RW + i * 16, 16)
            sl2 = pl.ds(i * 16, 16)
            d = dst_s[0, sl]
            dst2[j, sl2] = d
            idx2[j, sl2] = d * _R + rel_s[0, sl]
        gathers.append(
            pltpu.async_copy(tbl_hbm.at[idx2.at[j]], vals.at[j], sem))

    @pl.when(s == _NT - 1)
    def _mask_tail():
        for j in range(_TAILR, _ROWS):
            for i in range(_RW // 16):
                sl2 = pl.ds(i * 16, 16)
                dst2[j, sl2] = jnp.full((16,), _N, jnp.int32)

    for h in gathers:
        h.wait()
    plsc.subcore_barrier()
    for j in range(_ROWS):
        pltpu.sync_copy(vals.at[j], acc.at[dst2.at[j]], add=True)
    plsc.subcore_barrier()

    @pl.when(s < _NT - 1)
    def _full_out():
        pltpu.sync_copy(acc.at[pl.ds(s * _CH, _CH)], obuf)
        pltpu.sync_copy(obuf, out_hbm.at[pl.ds(s * _CH, _CH)])

    @pl.when(s == _NT - 1)
    def _tail_out():
        tail = _TAILR * _RW
        pltpu.sync_copy(acc.at[pl.ds(s * _CH, tail)], obuf.at[pl.ds(0, tail)])
        pltpu.sync_copy(obuf.at[pl.ds(0, tail)], out_hbm.at[pl.ds(s * _CH, tail)])


_sc_kernel = functools.partial(
    pl.kernel,
    out_type=jax.ShapeDtypeStruct((_N,), jnp.float32),
    mesh=plsc.VectorSubcoreMesh(
        core_axis_name="c", subcore_axis_name="s", num_cores=1),
    scratch_types=[
        pltpu.VMEM((1, _CH), jnp.int32),
        pltpu.VMEM((1, _CH), jnp.int32),
        pltpu.VMEM((_ROWS, _RW), jnp.int32),
        pltpu.VMEM((_ROWS, _RW), jnp.int32),
        pltpu.VMEM((_ROWS, _RW), jnp.float32),
        pltpu.VMEM((_CH,), jnp.float32),
        pltpu.VMEM((_CH,), jnp.float32),
        pltpu.VMEM_SHARED((_NP + 16,), jnp.float32),
        pltpu.SemaphoreType.DMA,
    ],
)(_sc_node_acc)


def _tc_body(x_ref, w_ref, a_ref, o_ref):
    o_ref[0] = (
        jnp.dot(x_ref[0], w_ref[...], preferred_element_type=jnp.float32)
        + a_ref[...]
    )


def kernel(X, edges, W, W0, inv_norm_constant):
    b, n, d = X.shape
    tbl = inv_norm_constant.reshape(-1)
    node_acc = _sc_kernel(edges, tbl)
    acc_col = node_acc[:, None]

    out = pl.pallas_call(
        _tc_body,
        out_shape=jax.ShapeDtypeStruct((b, n, d), jnp.float32),
        grid=(n // _RB,),
        in_specs=[
            pl.BlockSpec((1, _RB, d), lambda i: (0, i, 0)),
            pl.BlockSpec((d, d), lambda i: (0, 0)),
            pl.BlockSpec((_RB, 1), lambda i: (i, 0)),
        ],
        out_specs=pl.BlockSpec((1, _RB, d), lambda i: (0, i, 0)),
    )(X, W0, acc_col)
    return out
